# Initial kernel scaffold; baseline (speedup 1.0000x reference)
import jax
import jax.numpy as jnp
from jax.experimental import pallas as pl


def kernel(x, seq_vectors):
    raise NotImplementedError("write your pallas kernel here")


# speedup vs baseline: 4.6776x; 4.6776x over previous
import functools

import jax
import jax.numpy as jnp
import numpy as np
from jax import lax
from jax.experimental import pallas as pl
from jax.experimental.pallas import tpu as pltpu
from jax.experimental.pallas import tpu_sc as plsc

SEQ_VEC_SHAPE = 32
MAX_SEQ_LENGTH = 200
VOCAB_ROWS = 100000
BATCH = 4096

_NC = 2
_NS = 16
_NW = _NC * _NS

_TOTAL_ROWS = BATCH * MAX_SEQ_LENGTH
_ROWS_PER_W = _TOTAL_ROWS // _NW
_CHUNK = 4 * MAX_SEQ_LENGTH
_NCHUNK = _ROWS_PER_W // _CHUNK
_SCALE = float(np.sqrt(float(SEQ_VEC_SHAPE)))
_D = SEQ_VEC_SHAPE


def _pos_table() -> np.ndarray:
    initial_positions = np.arange(MAX_SEQ_LENGTH)[:, np.newaxis]
    positions = np.repeat(initial_positions, SEQ_VEC_SHAPE, axis=1)
    angle_rads = positions * (1.0 / 1000.0)
    s = np.sin(angle_rads)[::2]
    c = 1.0 - np.cos(angle_rads)[1::2]
    return np.vstack([s, c]).astype(np.float32)


_POS = _pos_table()


@functools.partial(
    pl.kernel,
    mesh=plsc.VectorSubcoreMesh(core_axis_name="c", subcore_axis_name="s"),
    out_type=jax.ShapeDtypeStruct((_TOTAL_ROWS * _D,), jnp.float32),
    scratch_types=[
        pltpu.VMEM((_CHUNK,), jnp.int32),
        pltpu.VMEM((_CHUNK, _D), jnp.float32),
        pltpu.VMEM((_CHUNK * _D,), jnp.float32),
        pltpu.VMEM((MAX_SEQ_LENGTH * _D,), jnp.float32),
        pltpu.SemaphoreType.DMA,
    ],
    compiler_params=pltpu.CompilerParams(use_tc_tiling_on_sc=False),
)
def _sc_embed(idx_hbm, table_hbm, pos_hbm, out_hbm, idx_v, rows_v, out_v, pos_v, sem):
    wid = lax.axis_index("s") * _NC + lax.axis_index("c")
    wbase = wid * _ROWS_PER_W
    table2d = table_hbm

    pltpu.sync_copy(pos_hbm, pos_v)

    def chunk_body(c, carry):
        base = wbase + c * _CHUNK
        pltpu.sync_copy(idx_hbm.at[pl.ds(base, _CHUNK)], idx_v)
        pltpu.async_copy(table2d.at[idx_v], rows_v, sem).wait()

        def row_body(t, carry2):
            for h in range(_D // 16):
                pv = pos_v[pl.ds(t * _D + h * 16, 16)]
                for p in range(_CHUNK // MAX_SEQ_LENGTH):
                    r = p * MAX_SEQ_LENGTH + t
                    out_v[pl.ds(r * _D + h * 16, 16)] = (
                        rows_v[r, pl.ds(h * 16, 16)] * _SCALE + pv
                    )
            return carry2

        lax.fori_loop(0, MAX_SEQ_LENGTH, row_body, 0)
        pltpu.sync_copy(out_v, out_hbm.at[pl.ds(base * _D, _CHUNK * _D)])
        return carry

    lax.fori_loop(0, _NCHUNK, chunk_body, 0)


def kernel(x, seq_vectors):
    flat_idx = x.reshape(_TOTAL_ROWS)
    pos = jnp.asarray(_POS).reshape(-1)
    table_flat = seq_vectors
    out = _sc_embed(flat_idx, table_flat, pos)
    return out.reshape(BATCH, MAX_SEQ_LENGTH, SEQ_VEC_SHAPE)
